# baseline (speedup 1.0000x reference)
import jax
import jax.numpy as jnp
from jax.experimental import pallas as pl
from jax.experimental.pallas import tpu as pltpu

B, S, D = 4, 2048, 1024


def _body(x_hbm, emb_hbm, o_hbm, eb, xb0, xb1, xb2, xb3,
          se, si0, si1, si2, si3, so0, so1, so2, so3):
    xbufs = (xb0, xb1, xb2, xb3)
    si = (si0, si1, si2, si3)
    so = (so0, so1, so2, so3)

    def xcopy(b):
        return pltpu.make_async_copy(x_hbm.at[b], xbufs[b], si[b])

    def ocopy(b):
        return pltpu.make_async_copy(xbufs[b], o_hbm.at[b], so[b])

    ecopy = pltpu.make_async_copy(emb_hbm, eb, se)
    ecopy.start()
    for b in range(B):
        xcopy(b).start()
    ecopy.wait()
    for b in range(B):
        xcopy(b).wait()
        xb = xbufs[b]
        xb[...] = xb[...] + eb[...]
        ocopy(b).start()
    for b in range(B):
        ocopy(b).wait()


def kernel(x, emb_weight):
    return pl.pallas_call(
        _body,
        in_specs=[
            pl.BlockSpec(memory_space=pl.ANY),
            pl.BlockSpec(memory_space=pl.ANY),
        ],
        out_specs=pl.BlockSpec(memory_space=pl.ANY),
        out_shape=jax.ShapeDtypeStruct(x.shape, x.dtype),
        scratch_shapes=(
            [pltpu.VMEM((S, D), jnp.float32) for _ in range(5)]
            + [pltpu.SemaphoreType.DMA for _ in range(9)]
        ),
    )(x, emb_weight)
